# baseline (speedup 1.0000x reference)
import jax
import jax.numpy as jnp
from jax.experimental import pallas as pl
from jax.experimental.pallas import tpu as pltpu

N = 10000
BM = 400
NB = N // BM
CW = 1664
NCH = 6
NTAIL = N - CW * NCH
NZ = 84
S_Z = NB
S_F = NB + NZ
S_R = S_F + 1
GRID = S_R + NB


def _xw_kernel(x_ref, w1_ref, o_ref):
    o_ref[...] = jnp.dot(x_ref[...], w1_ref[...],
                         preferred_element_type=jnp.float32)


def _zchunk(sp):
    g = ((sp >= 24).astype(jnp.int32) + (sp >= 44).astype(jnp.int32)
         + (sp >= 60).astype(jnp.int32) + (sp >= 72).astype(jnp.int32)
         + (sp >= 80).astype(jnp.int32))
    base = jnp.where(g == 0, 0, jnp.where(g == 1, 24, jnp.where(
        g == 2, 44, jnp.where(g == 3, 60, jnp.where(g == 4, 72, 80)))))
    n = 6 - g
    local = sp - base
    return 4 * g + local // n, g + local % n


def _mega_kernel(xw_ref, adj_ref, b1_ref, w2_ref, b2_ref, tail_ref,
                 o_ref,
                 fbuf0, fbuf1, zbuf0, zbuf1, hw_ref, zacc_ref, znorm_ref,
                 in_sem, zr_sem, ow_sem):
    s = pl.program_id(0)
    fbufs = (fbuf0, fbuf1)
    zbufs = (zbuf0, zbuf1)

    @pl.when(s == 0)
    def _boot():
        hw_ref[...] = jnp.zeros(hw_ref.shape, hw_ref.dtype)
        pltpu.make_async_copy(adj_ref.at[pl.ds(0, BM), :], fbuf0,
                              in_sem.at[0]).start()
        pltpu.make_async_copy(adj_ref.at[pl.ds(BM, BM), :], fbuf1,
                              in_sem.at[1]).start()

    @pl.when(s < S_Z)
    def _h_phase():
        i = s
        slot = jax.lax.rem(i, 2)
        for b_ in (0, 1):
            @pl.when(slot == b_)
            def _h_slot(b_=b_):
                fb = fbufs[b_]
                pltpu.make_async_copy(adj_ref.at[pl.ds(i * BM, BM), :],
                                      fb, in_sem.at[b_]).wait()
                acc = jnp.dot(fb[...], xw_ref[...],
                              preferred_element_type=jnp.float32)
                h = jnp.maximum(acc + b1_ref[...], 0.0)
                hw_ref[pl.ds(i * BM, BM), :] = jnp.dot(
                    h, w2_ref[...], preferred_element_type=jnp.float32)
                bcols = ((i + 1) * BM) // CW * CW
                cols = jax.lax.broadcasted_iota(jnp.int32, (BM, N), 1)
                fb[...] = jnp.where(cols < bcols, fb[...], 0.0)
                zacc_ref[pl.ds(i * BM, BM), :] = jnp.dot(
                    fb[...], hw_ref[...],
                    preferred_element_type=jnp.float32)

                @pl.when(i + 2 < NB)
                def _next():
                    pltpu.make_async_copy(
                        adj_ref.at[pl.ds((i + 2) * BM, BM), :],
                        fb, in_sem.at[b_]).start()

    @pl.when(s == S_Z)
    def _zboot():
        pltpu.make_async_copy(adj_ref.at[pl.ds(0, BM), pl.ds(0, CW)],
                              zbuf0, zr_sem.at[0]).start()
        pltpu.make_async_copy(adj_ref.at[pl.ds(0, BM), pl.ds(CW, CW)],
                              zbuf1, zr_sem.at[1]).start()

    @pl.when((s >= S_Z) & (s < S_F))
    def _z_phase():
        sp = s - S_Z
        i, c = _zchunk(sp)
        slot = jax.lax.rem(sp, 2)
        for b_ in (0, 1):
            @pl.when(slot == b_)
            def _z_slot(b_=b_):
                zb = zbufs[b_]
                pltpu.make_async_copy(
                    adj_ref.at[pl.ds(i * BM, BM), pl.ds(c * CW, CW)],
                    zb, zr_sem.at[b_]).wait()
                part = jnp.dot(zb[...], hw_ref[pl.ds(c * CW, CW), :],
                               preferred_element_type=jnp.float32)
                zacc_ref[pl.ds(i * BM, BM), :] = (
                    zacc_ref[pl.ds(i * BM, BM), :] + part)

                @pl.when(sp + 2 < NZ)
                def _next():
                    i2, c2 = _zchunk(sp + 2)
                    pltpu.make_async_copy(
                        adj_ref.at[pl.ds(i2 * BM, BM),
                                   pl.ds(c2 * CW, CW)],
                        zb, zr_sem.at[b_]).start()

    @pl.when(s == S_F)
    def _f_phase():
        ht = hw_ref[pl.ds(CW * NCH, NTAIL), :].astype(jnp.bfloat16)
        tail = jnp.dot(tail_ref[...], ht,
                       preferred_element_type=jnp.float32)
        g = zacc_ref[...] + tail + b2_ref[...]
        nrm = jnp.sqrt(jnp.sum(g * g, axis=1, keepdims=True))
        znorm_ref[...] = (g / (nrm + 1e-12)).astype(jnp.bfloat16)

    @pl.when(s >= S_R)
    def _r_phase():
        r = s - S_R
        slot = jax.lax.rem(r, 2)
        for b_ in (0, 1):
            @pl.when(slot == b_)
            def _r_slot(b_=b_):
                fb = fbufs[b_]

                @pl.when(r >= 2)
                def _drain():
                    pltpu.make_async_copy(
                        fb, o_ref.at[pl.ds((r - 2) * BM, BM), :],
                        ow_sem.at[b_]).wait()

                prod = jax.lax.dot_general(
                    znorm_ref[pl.ds(r * BM, BM), :], znorm_ref[...],
                    dimension_numbers=(((1,), (1,)), ((), ())),
                    preferred_element_type=jnp.float32)
                fb[...] = jax.nn.sigmoid(prod)
                pltpu.make_async_copy(fb, o_ref.at[pl.ds(r * BM, BM), :],
                                      ow_sem.at[b_]).start()

        @pl.when(r == NB - 1)
        def _final_drain():
            pltpu.make_async_copy(fbufs[1],
                                  o_ref.at[pl.ds((NB - 2) * BM, BM), :],
                                  ow_sem.at[1]).wait()
            pltpu.make_async_copy(fbufs[0],
                                  o_ref.at[pl.ds((NB - 1) * BM, BM), :],
                                  ow_sem.at[0]).wait()


def kernel(x, adj, W1, b1, W2, b2):
    b1 = b1.reshape(1, -1)
    b2 = b2.reshape(1, -1)
    nfeat = W1.shape[0]
    nhid = W1.shape[1]
    ndim = W2.shape[1]

    xw = pl.pallas_call(
        _xw_kernel,
        out_shape=jax.ShapeDtypeStruct((N, nhid), jnp.float32),
    )(x, W1)

    adj_tail = adj[:, CW * NCH:].astype(jnp.bfloat16)

    recon = pl.pallas_call(
        _mega_kernel,
        grid=(GRID,),
        in_specs=[
            pl.BlockSpec((N, nhid), lambda s: (0, 0)),
            pl.BlockSpec(memory_space=pltpu.MemorySpace.HBM),
            pl.BlockSpec((1, nhid), lambda s: (0, 0)),
            pl.BlockSpec((nhid, ndim), lambda s: (0, 0)),
            pl.BlockSpec((1, ndim), lambda s: (0, 0)),
            pl.BlockSpec((N, NTAIL), lambda s: (0, 0)),
        ],
        out_specs=pl.BlockSpec(memory_space=pltpu.MemorySpace.HBM),
        out_shape=jax.ShapeDtypeStruct((N, N), jnp.float32),
        scratch_shapes=[
            pltpu.VMEM((BM, N), jnp.float32),
            pltpu.VMEM((BM, N), jnp.float32),
            pltpu.VMEM((BM, CW), jnp.float32),
            pltpu.VMEM((BM, CW), jnp.float32),
            pltpu.VMEM((N, ndim), jnp.float32),
            pltpu.VMEM((N, ndim), jnp.float32),
            pltpu.VMEM((N, ndim), jnp.bfloat16),
            pltpu.SemaphoreType.DMA((2,)),
            pltpu.SemaphoreType.DMA((2,)),
            pltpu.SemaphoreType.DMA((2,)),
        ],
        compiler_params=pltpu.CompilerParams(
            dimension_semantics=("arbitrary",),
            vmem_limit_bytes=100 * 1024 * 1024,
        ),
    )(xw, adj, b1, W2, b2, adj_tail)

    return recon
